# Initial kernel scaffold; baseline (speedup 1.0000x reference)
import jax
import jax.numpy as jnp
from jax.experimental import pallas as pl


def kernel(pois_embs, pad_all_train_sessions, HG_up, HG_pu):
    raise NotImplementedError("write your pallas kernel here")


# speedup vs baseline: 1.0137x; 1.0137x over previous
import functools

import jax
import jax.numpy as jnp
from jax.experimental import pallas as pl
from jax.experimental.pallas import tpu as pltpu

P = 10000
U = 10000
D = 128
BM = 200
NB_UP = U // BM
NB_PU = P // BM


def _fused_body(up_ref, pu_ref, pe_ref, out_ref, tmp_ref):
    i = pl.program_id(0)

    @pl.when(i < NB_UP)
    def _phase1():
        blk = jax.lax.dot_general(
            up_ref[...], pe_ref[...],
            dimension_numbers=(((1,), (0,)), ((), ())),
            preferred_element_type=jnp.float32,
        )
        tmp_ref[pl.ds(i * BM, BM), :] = blk

    @pl.when(i >= NB_UP)
    def _phase2():
        out_ref[...] = jax.lax.dot_general(
            pu_ref[...], tmp_ref[...],
            dimension_numbers=(((1,), (0,)), ((), ())),
            preferred_element_type=jnp.float32,
        )


@jax.jit
def _fused(pois_embs, HG_up, HG_pu):
    grid = (NB_UP + NB_PU,)
    return pl.pallas_call(
        _fused_body,
        grid=grid,
        in_specs=[
            pl.BlockSpec((BM, P), lambda i: (jnp.minimum(i, NB_UP - 1), 0)),
            pl.BlockSpec((BM, U), lambda i: (jnp.maximum(i - NB_UP, 0), 0)),
            pl.BlockSpec((P, D), lambda i: (0, 0)),
        ],
        out_specs=pl.BlockSpec((BM, D), lambda i: (jnp.maximum(i - NB_UP, 0), 0)),
        out_shape=jax.ShapeDtypeStruct((P, D), jnp.float32),
        scratch_shapes=[pltpu.VMEM((U, D), jnp.float32)],
        compiler_params=pltpu.CompilerParams(
            dimension_semantics=("arbitrary",),
        ),
    )(HG_up, HG_pu, pois_embs)


def kernel(pois_embs, pad_all_train_sessions, HG_up, HG_pu):
    del pad_all_train_sessions
    return _fused(pois_embs, HG_up, HG_pu)
